# Initial kernel scaffold; baseline (speedup 1.0000x reference)
import jax
import jax.numpy as jnp
from jax.experimental import pallas as pl


def kernel(pos, pin_pos, node_size_x, node_size_y, netpin_start, flat_netpin):
    raise NotImplementedError("write your pallas kernel here")


# speedup vs baseline: 1.0015x; 1.0015x over previous
import functools

import jax
import jax.numpy as jnp
from jax import lax
from jax.experimental import pallas as pl
from jax.experimental.pallas import tpu as pltpu

NUM_BINS = 256
XL, XH, YL, YH = 0.0, 1024.0, 0.0, 1024.0
NUM_NETS = 16384
NUM_NODES = 20000
NUM_MOVABLE = 16384
NUM_PINS = 65536
BIN = (XH - XL) / NUM_BINS
BIN_AREA = BIN * BIN
CAP_H = 0.1
CAP_V = 0.1
MAX_RATE = 2.0
MIN_RATE = 0.5

TN = 2048
NT = NUM_NETS // TN


def _demand_body(xm_ref, xM_ref, ym_ref, yM_ref, cnt_ref, rt_ref, acc_ref):
    i = pl.program_id(0)

    @pl.when(i == 0)
    def _():
        acc_ref[...] = jnp.zeros_like(acc_ref)

    valid = cnt_ref[0] > 0
    xm = jnp.where(valid, xm_ref[0], 0.0)
    xM = jnp.where(valid, xM_ref[0], 0.0)
    ym = jnp.where(valid, ym_ref[0], 0.0)
    yM = jnp.where(valid, yM_ref[0], 0.0)
    w = xM - xm
    h = yM - ym
    area = w * h
    pos = area > 0
    safe = jnp.where(pos, area, 1.0)
    dh = jnp.where(pos, w / safe, 0.0)
    dv = jnp.where(pos, h / safe, 0.0)
    b_lo = lax.broadcasted_iota(jnp.int32, (NUM_BINS, TN), 0).astype(jnp.float32) * BIN
    ox = jnp.clip(jnp.minimum(xM, b_lo + BIN) - jnp.maximum(xm, b_lo), 0.0, BIN)
    oy = jnp.clip(jnp.minimum(yM, b_lo + BIN) - jnp.maximum(ym, b_lo), 0.0, BIN)
    stacked = jnp.concatenate([ox * dh, ox * dv], axis=0)
    acc_ref[...] += lax.dot_general(
        stacked, oy, (((1,), (1,)), ((), ())),
        preferred_element_type=jnp.float32)

    @pl.when(i == NT - 1)
    def _():
        util = acc_ref[...] / (CAP_H * BIN_AREA)
        rt_ref[...] = jnp.clip(
            jnp.maximum(util[:NUM_BINS, :], util[NUM_BINS:, :]),
            MIN_RATE, MAX_RATE)


def _instance_body(rt_ref, nx_ref, ny_ref, sx_ref, sy_ref, out_ref):
    b_lo = lax.broadcasted_iota(jnp.int32, (NUM_BINS, TN), 0).astype(jnp.float32) * BIN
    nx = nx_ref[0]
    ny = ny_ref[0]
    nox = jnp.clip(jnp.minimum(nx + sx_ref[0], b_lo + BIN)
                   - jnp.maximum(nx, b_lo), 0.0, BIN)
    noy = jnp.clip(jnp.minimum(ny + sy_ref[0], b_lo + BIN)
                   - jnp.maximum(ny, b_lo), 0.0, BIN)
    t1 = lax.dot_general(rt_ref[...], nox, (((0,), (0,)), ((), ())),
                         preferred_element_type=jnp.float32)
    out_ref[0] = jnp.sum(t1 * noy, axis=0, keepdims=True)


def _dense_pipeline(x_min, x_max, y_min, y_max, counts, nx, ny, sx, sy):
    r2 = lambda a: a.reshape(NT, 1, TN)
    rt = pl.pallas_call(
        _demand_body,
        grid=(NT,),
        in_specs=[pl.BlockSpec((1, 1, TN), lambda i: (i, 0, 0))] * 5,
        out_specs=pl.BlockSpec((NUM_BINS, NUM_BINS), lambda i: (0, 0)),
        out_shape=jax.ShapeDtypeStruct((NUM_BINS, NUM_BINS), jnp.float32),
        scratch_shapes=[pltpu.VMEM((2 * NUM_BINS, NUM_BINS), jnp.float32)],
    )(r2(x_min), r2(x_max), r2(y_min), r2(y_max), r2(counts))
    out = pl.pallas_call(
        _instance_body,
        grid=(NT,),
        in_specs=[pl.BlockSpec((NUM_BINS, NUM_BINS), lambda i: (0, 0))]
        + [pl.BlockSpec((1, 1, TN), lambda i: (i, 0, 0))] * 4,
        out_specs=pl.BlockSpec((1, 1, TN), lambda i: (i, 0, 0)),
        out_shape=jax.ShapeDtypeStruct((NT, 1, TN), jnp.float32),
    )(rt, r2(nx), r2(ny), r2(sx), r2(sy))
    return out.reshape(NUM_MOVABLE)


def kernel(pos, pin_pos, node_size_x, node_size_y, netpin_start, flat_netpin):
    pin_x = pin_pos[:NUM_PINS]
    pin_y = pin_pos[NUM_PINS:]
    counts = netpin_start[1:] - netpin_start[:-1]
    seg_ids = jnp.repeat(jnp.arange(NUM_NETS), counts,
                         total_repeat_length=NUM_PINS)
    px = jnp.take(pin_x, flat_netpin)
    py = jnp.take(pin_y, flat_netpin)
    x_min = jax.ops.segment_min(px, seg_ids, num_segments=NUM_NETS)
    x_max = jax.ops.segment_max(px, seg_ids, num_segments=NUM_NETS)
    y_min = jax.ops.segment_min(py, seg_ids, num_segments=NUM_NETS)
    y_max = jax.ops.segment_max(py, seg_ids, num_segments=NUM_NETS)

    nx = pos[:NUM_MOVABLE]
    ny = pos[NUM_NODES:NUM_NODES + NUM_MOVABLE]
    sx = node_size_x[:NUM_MOVABLE]
    sy = node_size_y[:NUM_MOVABLE]
    return _dense_pipeline(x_min, x_max, y_min, y_max,
                           counts.astype(jnp.float32), nx, ny, sx, sy)
